# Initial kernel scaffold; baseline (speedup 1.0000x reference)
import jax
import jax.numpy as jnp
from jax.experimental import pallas as pl


def kernel(input, embedded, pos_emb):
    raise NotImplementedError("write your pallas kernel here")


# speedup vs baseline: 4.1494x; 4.1494x over previous
import jax
import jax.numpy as jnp
from jax import lax
from jax.experimental import pallas as pl
from jax.experimental.pallas import tpu as pltpu
from jax.experimental.pallas import tpu_sc as plsc

_B, _L, _D = 4096, 200, 64
_N = _B * _L
_NC, _NS = 2, 16
_NW = _NC * _NS
_K = 4
_C = _K * 128
_CHUNKS = _N // (_NW * _C)


def _sc_body(idx_hbm, emb_hbm, tab_hbm, out_hbm, idx_v, pe_v, emb_v, gsem):
    wid = lax.axis_index("s") * _NC + lax.axis_index("c")

    def chunk(i, carry):
        r0 = (wid * _CHUNKS + i) * _K
        base = r0 * 128
        pltpu.sync_copy(idx_hbm.at[pl.ds(r0, _K)], idx_v)
        copies = [
            pltpu.async_copy(tab_hbm.at[idx_v.at[j]],
                             pe_v.at[pl.ds(j * 128, 128)], gsem)
            for j in range(_K)
        ]
        pltpu.sync_copy(emb_hbm.at[pl.ds(base, _C)], emb_v)
        pltpu.sync_copy(emb_v, out_hbm.at[pl.ds(base, _C), pl.ds(0, _D)])
        for c in copies:
            c.wait()
        pltpu.sync_copy(pe_v, out_hbm.at[pl.ds(base, _C), pl.ds(_D, _D)])
        return carry

    lax.fori_loop(0, _CHUNKS, chunk, 0)


def kernel(input, embedded, pos_emb):
    idx = input.reshape(_N // 128, 128).astype(jnp.int32)
    emb = embedded.reshape(_N, _D)
    mesh = plsc.VectorSubcoreMesh(core_axis_name="c", subcore_axis_name="s")
    out = pl.kernel(
        _sc_body,
        out_type=jax.ShapeDtypeStruct((_N, 2 * _D), jnp.float32),
        mesh=mesh,
        scratch_types=[
            pltpu.VMEM((_K, 128), jnp.int32),
            pltpu.VMEM((_C, _D), jnp.float32),
            pltpu.VMEM((_C, _D), jnp.float32),
            pltpu.SemaphoreType.DMA,
        ],
        compiler_params=pltpu.CompilerParams(use_tc_tiling_on_sc=False),
    )(idx, emb, pos_emb)
    return out.reshape(_B, _L, 2 * _D)
